# Initial kernel scaffold; baseline (speedup 1.0000x reference)
import jax
import jax.numpy as jnp
from jax.experimental import pallas as pl


def kernel(x, pos_x, pos_y, levels, W):
    raise NotImplementedError("write your pallas kernel here")


# speedup vs baseline: 4.3862x; 4.3862x over previous
import jax
import jax.numpy as jnp
from jax import lax
from jax.experimental import pallas as pl
from jax.experimental.pallas import tpu as pltpu

B = 8
SIZE = 28
P = SIZE * SIZE
L = 256
D = 10000
DP = 10240
CD = 1024
NCHUNK = DP // CD
NC = 10


def _body(xf_ref, px_ref, py_ref, lev_ref, w_ref, out_ref):
    g = pl.program_id(0)

    idx = jnp.clip(jnp.round(xf_ref[...] * (L - 1)), 0, L - 1).astype(jnp.int32)

    lgrid = lax.broadcasted_iota(jnp.int32, (B, L, P), 1)
    oh = (idx[:, None, :] == lgrid).astype(jnp.bfloat16).reshape(B * L, P)

    s = py_ref[...][:, None, :] + px_ref[...][None, :, :]
    posmat = jnp.where(s > 0, 1.0, -1.0).astype(jnp.bfloat16).reshape(P, CD)

    acc = lax.dot_general(
        oh, posmat,
        dimension_numbers=(((1,), (0,)), ((), ())),
        preferred_element_type=jnp.float32,
    ).reshape(B, L, CD)

    ms = jnp.sum(acc * lev_ref[...][None, :, :], axis=1)
    enc = jnp.where(ms > 0, 1.0, -1.0).astype(jnp.float32)

    part = lax.dot_general(
        enc, w_ref[...],
        dimension_numbers=(((1,), (1,)), ((), ())),
        preferred_element_type=jnp.float32,
    )

    @pl.when(g == 0)
    def _():
        out_ref[...] = jnp.zeros_like(out_ref)

    out_ref[...] += part


def kernel(x, pos_x, pos_y, levels, W):
    xf = x.reshape(B, P)
    pad = DP - D
    px = jnp.pad(pos_x, ((0, 0), (0, pad)))
    py = jnp.pad(pos_y, ((0, 0), (0, pad)))
    lev = jnp.pad(levels, ((0, 0), (0, pad)))
    wp = jnp.pad(W, ((0, 0), (0, pad)))

    return pl.pallas_call(
        _body,
        grid=(NCHUNK,),
        in_specs=[
            pl.BlockSpec((B, P), lambda g: (0, 0)),
            pl.BlockSpec((SIZE, CD), lambda g: (0, g)),
            pl.BlockSpec((SIZE, CD), lambda g: (0, g)),
            pl.BlockSpec((L, CD), lambda g: (0, g)),
            pl.BlockSpec((NC, CD), lambda g: (0, g)),
        ],
        out_specs=pl.BlockSpec((B, NC), lambda g: (0, 0)),
        out_shape=jax.ShapeDtypeStruct((B, NC), jnp.float32),
        compiler_params=pltpu.CompilerParams(
            dimension_semantics=("arbitrary",),
        ),
    )(xf, px, py, lev, wp)


# speedup vs baseline: 5.5362x; 1.2622x over previous
import jax
import jax.numpy as jnp
from jax import lax
from jax.experimental import pallas as pl
from jax.experimental.pallas import tpu as pltpu

B = 8
SIZE = 28
P = SIZE * SIZE
L = 256
D = 10000
CD = 1024
NCHUNK = (D + CD - 1) // CD
NC = 10


def _body(xf_ref, px_ref, py_ref, lev_ref, w_ref, out_ref):
    g = pl.program_id(0)

    dmask = (g * CD + lax.broadcasted_iota(jnp.int32, (1, CD), 1)) < D

    idx = jnp.clip(jnp.round(xf_ref[...] * (L - 1)), 0, L - 1).astype(jnp.int32)

    lgrid = lax.broadcasted_iota(jnp.int32, (B, L, P), 1)
    oh = (idx[:, None, :] == lgrid).astype(jnp.int8).reshape(B * L, P)

    s = py_ref[...][:, None, :] + px_ref[...][None, :, :]
    posmat = jnp.where(s > 0, 1, -1).astype(jnp.int8).reshape(P, CD)

    acc = lax.dot_general(
        oh, posmat,
        dimension_numbers=(((1,), (0,)), ((), ())),
        preferred_element_type=jnp.int32,
    ).reshape(B, L, CD).astype(jnp.float32)

    lev = jnp.where(dmask, lev_ref[...], 0.0)
    ms = jnp.sum(acc * lev[None, :, :], axis=1)
    enc = jnp.where(ms > 0, 1.0, -1.0).astype(jnp.float32)

    wm = jnp.where(dmask, w_ref[...], 0.0)
    part = lax.dot_general(
        enc, wm,
        dimension_numbers=(((1,), (1,)), ((), ())),
        preferred_element_type=jnp.float32,
    )

    @pl.when(g == 0)
    def _():
        out_ref[...] = jnp.zeros_like(out_ref)

    out_ref[...] += part


def kernel(x, pos_x, pos_y, levels, W):
    xf = x.reshape(B, P)
    return pl.pallas_call(
        _body,
        grid=(NCHUNK,),
        in_specs=[
            pl.BlockSpec((B, P), lambda g: (0, 0)),
            pl.BlockSpec((SIZE, CD), lambda g: (0, g)),
            pl.BlockSpec((SIZE, CD), lambda g: (0, g)),
            pl.BlockSpec((L, CD), lambda g: (0, g)),
            pl.BlockSpec((NC, CD), lambda g: (0, g)),
        ],
        out_specs=pl.BlockSpec((B, NC), lambda g: (0, 0)),
        out_shape=jax.ShapeDtypeStruct((B, NC), jnp.float32),
        compiler_params=pltpu.CompilerParams(
            dimension_semantics=("arbitrary",),
        ),
    )(xf, pos_x, pos_y, levels, W)


# speedup vs baseline: 6.3353x; 1.1443x over previous
import jax
import jax.numpy as jnp
from jax import lax
from jax.experimental import pallas as pl
from jax.experimental.pallas import tpu as pltpu

B = 8
SIZE = 28
P = SIZE * SIZE
L = 256
D = 10000
CD = 2048
NCHUNK = (D + CD - 1) // CD
NC = 10


def _body(xf_ref, px_ref, py_ref, lev_ref, w_ref, out_ref):
    g = pl.program_id(0)

    dmask = (g * CD + lax.broadcasted_iota(jnp.int32, (1, CD), 1)) < D

    idx = jnp.clip(jnp.round(xf_ref[...] * (L - 1)), 0, L - 1).astype(jnp.int32)

    lgrid = lax.broadcasted_iota(jnp.int32, (B, L, P), 1)
    oh = (idx[:, None, :] == lgrid).astype(jnp.bfloat16).reshape(B * L, P)

    s = py_ref[...][:, None, :] + px_ref[...][None, :, :]
    posmat = jnp.where(s > 0, 1.0, -1.0).astype(jnp.bfloat16).reshape(P, CD)

    acc = lax.dot_general(
        oh, posmat,
        dimension_numbers=(((1,), (0,)), ((), ())),
        preferred_element_type=jnp.float32,
    ).reshape(B, L, CD)

    lev = jnp.where(dmask, lev_ref[...], 0.0)
    ms = jnp.sum(acc * lev[None, :, :], axis=1)
    enc = jnp.where(ms > 0, 1.0, -1.0).astype(jnp.float32)

    wm = jnp.where(dmask, w_ref[...], 0.0)
    part = lax.dot_general(
        enc, wm,
        dimension_numbers=(((1,), (1,)), ((), ())),
        preferred_element_type=jnp.float32,
    )

    @pl.when(g == 0)
    def _():
        out_ref[...] = jnp.zeros_like(out_ref)

    out_ref[...] += part


def kernel(x, pos_x, pos_y, levels, W):
    xf = x.reshape(B, P)
    return pl.pallas_call(
        _body,
        grid=(NCHUNK,),
        in_specs=[
            pl.BlockSpec((B, P), lambda g: (0, 0)),
            pl.BlockSpec((SIZE, CD), lambda g: (0, g)),
            pl.BlockSpec((SIZE, CD), lambda g: (0, g)),
            pl.BlockSpec((L, CD), lambda g: (0, g)),
            pl.BlockSpec((NC, CD), lambda g: (0, g)),
        ],
        out_specs=pl.BlockSpec((B, NC), lambda g: (0, 0)),
        out_shape=jax.ShapeDtypeStruct((B, NC), jnp.float32),
        compiler_params=pltpu.CompilerParams(
            dimension_semantics=("arbitrary",),
        ),
    )(xf, pos_x, pos_y, levels, W)
